# baseline (speedup 1.0000x reference)
import functools

import jax
import jax.numpy as jnp
from jax import lax
from jax.experimental import pallas as pl
from jax.experimental.pallas import tpu as pltpu
from jax.experimental.pallas import tpu_sc as plsc

N = 10000
E = 320000
FIN = 128
HID = 128
LZ = 32
LT = 64
THID = 256

NROW = 10240
EPAD = 327680
NW = 32
CHUNK = 128
CPT = EPAD // (NW * CHUNK)
RPT = NROW // 16
ZROWS = 64

def _deg_body(dst_hbm, out_hbm, ones, didx, ssem, hist):
    cid = lax.axis_index("c")
    sid = lax.axis_index("s")
    wid = sid * 2 + cid
    zero16 = jnp.zeros((16,), jnp.float32)
    ones16 = jnp.ones((16,), jnp.float32)
    base = sid * RPT

    pltpu.sync_copy(dst_hbm.at[pl.ds(wid * CPT, CPT)], didx)

    def _z(k, _):
        ones[k // (FIN // 16), pl.ds((k % (FIN // 16)) * 16, 16)] = zero16
        return 0

    lax.fori_loop(0, CHUNK * (FIN // 16), _z, 0)

    def _zc(k, _):
        pltpu.sync_copy(ones, hist.at[pl.ds(base + k * CHUNK, CHUNK)])
        return 0

    lax.fori_loop(0, RPT // CHUNK, _zc, 0)

    def _o(k, _):
        ones[k // (FIN // 16), pl.ds((k % (FIN // 16)) * 16, 16)] = ones16
        return 0

    lax.fori_loop(0, CHUNK * (FIN // 16), _o, 0)
    plsc.subcore_barrier()

    def _e(c, _):
        pltpu.async_copy(ones, hist.at[didx.at[c]], ssem, add=True)
        return 0

    lax.fori_loop(0, CPT, _e, 0)

    def _w(c, _):
        pltpu.make_async_copy(ones, hist.at[didx.at[c]], ssem).wait()
        return 0

    lax.fori_loop(0, CPT, _w, 0)
    plsc.subcore_barrier()
    pltpu.sync_copy(hist.at[pl.ds(base, RPT)], out_hbm.at[cid, pl.ds(base, RPT)])


G = 16
NGRP = CPT // G


def _msg_body(src_hbm, dst_hbm, hs_hbm, out_hbm, sidx, didx, rows,
              isem, g0, g1, ssem, acc):
    gsem = (g0, g1)
    cid = lax.axis_index("c")
    sid = lax.axis_index("s")
    wid = sid * 2 + cid
    zero16 = jnp.zeros((16,), jnp.float32)
    base = sid * RPT
    base0 = wid * CPT

    pltpu.sync_copy(src_hbm.at[pl.ds(base0, G)], sidx.at[0])
    pltpu.sync_copy(dst_hbm.at[pl.ds(base0, G)], didx.at[0])

    def _z(k, _):
        rows[0, k // (FIN // 16), pl.ds((k % (FIN // 16)) * 16, 16)] = zero16
        return 0

    lax.fori_loop(0, CHUNK * (FIN // 16), _z, 0)

    def _zc(k, _):
        pltpu.sync_copy(rows.at[0], acc.at[pl.ds(base + k * CHUNK, CHUNK)])
        return 0

    lax.fori_loop(0, RPT // CHUNK, _zc, 0)
    plsc.subcore_barrier()

    for b in range(2):
        pltpu.async_copy(hs_hbm.at[sidx.at[0, b]], rows.at[b], gsem[b])

    def _grp(g, _):
        slot = g % 2
        nxt = (g + 1) % 2

        def _rnd(rr, _):
            @pl.when((rr == 0) & (g + 1 < NGRP))
            def _():
                pltpu.async_copy(
                    src_hbm.at[pl.ds(base0 + (g + 1) * G, G)],
                    sidx.at[nxt], isem)
                pltpu.async_copy(
                    dst_hbm.at[pl.ds(base0 + (g + 1) * G, G)],
                    didx.at[nxt], isem)

            for b in range(2):
                k = 2 * rr + b
                pltpu.make_async_copy(
                    hs_hbm.at[sidx.at[slot, k]], rows.at[b], gsem[b]).wait()
                pltpu.async_copy(
                    rows.at[b], acc.at[didx.at[slot, k]], ssem, add=True)
            for b in range(2):
                pltpu.make_async_copy(
                    rows.at[b], acc.at[didx.at[slot, 2 * rr + b]], ssem).wait()

            @pl.when(rr < G // 2 - 1)
            def _():
                for b in range(2):
                    pltpu.async_copy(
                        hs_hbm.at[sidx.at[slot, 2 * rr + 2 + b]],
                        rows.at[b], gsem[b])

            @pl.when((rr == G // 2 - 1) & (g + 1 < NGRP))
            def _():
                pltpu.make_async_copy(
                    src_hbm.at[pl.ds(base0, G)], sidx.at[nxt], isem).wait()
                pltpu.make_async_copy(
                    dst_hbm.at[pl.ds(base0, G)], didx.at[nxt], isem).wait()
                for b in range(2):
                    pltpu.async_copy(
                        hs_hbm.at[sidx.at[nxt, b]], rows.at[b], gsem[b])
            return 0

        lax.fori_loop(0, G // 2, _rnd, 0)
        return 0

    lax.fori_loop(0, NGRP, _grp, 0)
    plsc.subcore_barrier()
    pltpu.sync_copy(acc.at[pl.ds(base, RPT)], out_hbm.at[cid, pl.ds(base, RPT)])


@functools.cache
def _sc_kernels():
    mesh = plsc.VectorSubcoreMesh(
        core_axis_name="c", subcore_axis_name="s", num_cores=2, num_subcores=16)
    deg = pl.kernel(
        _deg_body,
        out_type=jax.ShapeDtypeStruct((2, NROW, FIN), jnp.float32),
        mesh=mesh,
        scratch_types=[
            pltpu.VMEM((CHUNK, FIN), jnp.float32),
            pltpu.VMEM((CPT, CHUNK), jnp.int32),
            pltpu.SemaphoreType.DMA,
            pltpu.VMEM_SHARED((NROW, FIN), jnp.float32),
        ],
    )
    msg = pl.kernel(
        _msg_body,
        out_type=jax.ShapeDtypeStruct((2, NROW, FIN), jnp.float32),
        mesh=mesh,
        scratch_types=[
            pltpu.VMEM((2, G, CHUNK), jnp.int32),
            pltpu.VMEM((2, G, CHUNK), jnp.int32),
            pltpu.VMEM((2, CHUNK, FIN), jnp.float32),
            pltpu.SemaphoreType.DMA,
            pltpu.SemaphoreType.DMA,
            pltpu.SemaphoreType.DMA,
            pltpu.SemaphoreType.DMA,
            pltpu.VMEM_SHARED((NROW, FIN), jnp.float32),
        ],
    )
    return deg, msg


BR = 1000


def _gcn_pre_body(x_ref, w_ref, hist_ref, hs_ref, dinv_ref):
    hb = hist_ref[...]
    deg = hb[0, :, 0:1] + hb[1, :, 0:1] + 1.0
    dinv = lax.rsqrt(jnp.maximum(deg, 1.0))
    h = jnp.dot(x_ref[...], w_ref[...], preferred_element_type=jnp.float32)
    hs_ref[...] = h * dinv
    dinv_ref[...] = dinv


def _gcn_pre(x, w, hist):
    return pl.pallas_call(
        _gcn_pre_body,
        grid=(N // BR,),
        in_specs=[
            pl.BlockSpec((BR, FIN), lambda i: (i, 0)),
            pl.BlockSpec((FIN, HID), lambda i: (0, 0)),
            pl.BlockSpec((2, BR, FIN), lambda i: (0, i, 0)),
        ],
        out_specs=[
            pl.BlockSpec((BR, HID), lambda i: (i, 0)),
            pl.BlockSpec((BR, 1), lambda i: (i, 0)),
        ],
        out_shape=[
            jax.ShapeDtypeStruct((N, HID), jnp.float32),
            jax.ShapeDtypeStruct((N, 1), jnp.float32),
        ],
    )(x, w, hist)


def _agg_body(acc_ref, hs_ref, dinv_ref, b_ref, agg_ref, s_ref, q_ref):
    a = acc_ref[...]
    t = a[0] + a[1] + hs_ref[...]
    agg = t * dinv_ref[...] + b_ref[...]
    agg_ref[...] = agg
    ps = jnp.sum(agg, axis=0, keepdims=True)
    pq = jnp.sum(agg * agg, axis=0, keepdims=True)
    i = pl.program_id(0)

    @pl.when(i == 0)
    def _():
        s_ref[...] = ps
        q_ref[...] = pq

    @pl.when(i != 0)
    def _():
        s_ref[...] += ps
        q_ref[...] += pq


def _agg(acc, hs, dinv, b_gcn):
    return pl.pallas_call(
        _agg_body,
        grid=(N // BR,),
        in_specs=[
            pl.BlockSpec((2, BR, HID), lambda i: (0, i, 0)),
            pl.BlockSpec((BR, HID), lambda i: (i, 0)),
            pl.BlockSpec((BR, 1), lambda i: (i, 0)),
            pl.BlockSpec((1, HID), lambda i: (0, 0)),
        ],
        out_specs=[
            pl.BlockSpec((BR, HID), lambda i: (i, 0)),
            pl.BlockSpec((1, HID), lambda i: (0, 0)),
            pl.BlockSpec((1, HID), lambda i: (0, 0)),
        ],
        out_shape=[
            jax.ShapeDtypeStruct((N, HID), jnp.float32),
            jax.ShapeDtypeStruct((1, HID), jnp.float32),
            jax.ShapeDtypeStruct((1, HID), jnp.float32),
        ],
    )(acc, hs, dinv, b_gcn)


def _enc_body(agg_ref, s_ref, q_ref, g1_ref, b1_ref, wmu_ref, bmu_ref,
              wlv_ref, blv_ref, wp_ref, bp_ref,
              mu_ref, lv_ref, zp_ref, zs_ref, zq_ref):
    inv_n = 1.0 / N
    mean = s_ref[...] * inv_n
    var = q_ref[...] * inv_n - mean * mean
    scale = g1_ref[...] * lax.rsqrt(var + 1e-5)
    hact = jnp.maximum((agg_ref[...] - mean) * scale + b1_ref[...], 0.0)
    mu = jnp.dot(hact, wmu_ref[...], preferred_element_type=jnp.float32) + bmu_ref[...]
    mu_ref[...] = mu
    lv_ref[...] = jnp.dot(hact, wlv_ref[...], preferred_element_type=jnp.float32) + blv_ref[...]
    zp = jnp.dot(mu, wp_ref[...], preferred_element_type=jnp.float32) + bp_ref[...]
    zp_ref[...] = zp
    ps = jnp.sum(zp, axis=0, keepdims=True)
    pq = jnp.sum(zp * zp, axis=0, keepdims=True)
    i = pl.program_id(0)

    @pl.when(i == 0)
    def _():
        zs_ref[...] = ps
        zq_ref[...] = pq

    @pl.when(i != 0)
    def _():
        zs_ref[...] += ps
        zq_ref[...] += pq


def _enc(agg, ssum, ssq, gamma1, beta1, w_mu, b_mu, w_lv, b_lv, w_p, b_p):
    return pl.pallas_call(
        _enc_body,
        grid=(N // BR,),
        in_specs=[
            pl.BlockSpec((BR, HID), lambda i: (i, 0)),
            pl.BlockSpec((1, HID), lambda i: (0, 0)),
            pl.BlockSpec((1, HID), lambda i: (0, 0)),
            pl.BlockSpec((1, HID), lambda i: (0, 0)),
            pl.BlockSpec((1, HID), lambda i: (0, 0)),
            pl.BlockSpec((HID, LZ), lambda i: (0, 0)),
            pl.BlockSpec((1, LZ), lambda i: (0, 0)),
            pl.BlockSpec((HID, LZ), lambda i: (0, 0)),
            pl.BlockSpec((1, LZ), lambda i: (0, 0)),
            pl.BlockSpec((LZ, LT), lambda i: (0, 0)),
            pl.BlockSpec((1, LT), lambda i: (0, 0)),
        ],
        out_specs=[
            pl.BlockSpec((BR, LZ), lambda i: (i, 0)),
            pl.BlockSpec((BR, LZ), lambda i: (i, 0)),
            pl.BlockSpec((BR, LT), lambda i: (i, 0)),
            pl.BlockSpec((1, LT), lambda i: (0, 0)),
            pl.BlockSpec((1, LT), lambda i: (0, 0)),
        ],
        out_shape=[
            jax.ShapeDtypeStruct((N, LZ), jnp.float32),
            jax.ShapeDtypeStruct((N, LZ), jnp.float32),
            jax.ShapeDtypeStruct((N, LT), jnp.float32),
            jax.ShapeDtypeStruct((1, LT), jnp.float32),
            jax.ShapeDtypeStruct((1, LT), jnp.float32),
        ],
    )(agg, ssum, ssq, gamma1, beta1, w_mu, b_mu, w_lv, b_lv, w_p, b_p)


def _teach_body(zp_ref, zs_ref, zq_ref, gp_ref, bp_ref, wt1_ref, bt1_ref,
                wt2_ref, bt2_ref, xr_ref):
    inv_n = 1.0 / N
    mean = zs_ref[...] * inv_n
    var = zq_ref[...] * inv_n - mean * mean
    scale = gp_ref[...] * lax.rsqrt(var + 1e-5)
    zpn = (zp_ref[...] - mean) * scale + bp_ref[...]
    t = jnp.maximum(
        jnp.dot(zpn, wt1_ref[...], preferred_element_type=jnp.float32) + bt1_ref[...], 0.0)
    xr_ref[...] = jnp.dot(t, wt2_ref[...], preferred_element_type=jnp.float32) + bt2_ref[...]


def _teach(zp, zsum, zsq, gamma_p, beta_p, wt1, bt1, wt2, bt2):
    return pl.pallas_call(
        _teach_body,
        grid=(N // BR,),
        in_specs=[
            pl.BlockSpec((BR, LT), lambda i: (i, 0)),
            pl.BlockSpec((1, LT), lambda i: (0, 0)),
            pl.BlockSpec((1, LT), lambda i: (0, 0)),
            pl.BlockSpec((1, LT), lambda i: (0, 0)),
            pl.BlockSpec((1, LT), lambda i: (0, 0)),
            pl.BlockSpec((LT, THID), lambda i: (0, 0)),
            pl.BlockSpec((1, THID), lambda i: (0, 0)),
            pl.BlockSpec((THID, FIN), lambda i: (0, 0)),
            pl.BlockSpec((1, FIN), lambda i: (0, 0)),
        ],
        out_specs=pl.BlockSpec((BR, FIN), lambda i: (i, 0)),
        out_shape=jax.ShapeDtypeStruct((N, FIN), jnp.float32),
    )(zp, zsum, zsq, gamma_p, beta_p, wt1, bt1, wt2, bt2)


DR = 400


def _adj_body(zi_ref, zj_ref, out_ref):
    d = lax.dot_general(zi_ref[...], zj_ref[...], (((1,), (1,)), ((), ())),
                        preferred_element_type=jnp.float32)
    out_ref[...] = jax.nn.sigmoid(d)


def _adj(mu):
    return pl.pallas_call(
        _adj_body,
        grid=(N // DR,),
        in_specs=[
            pl.BlockSpec((DR, LZ), lambda i: (i, 0)),
            pl.BlockSpec((N, LZ), lambda i: (0, 0)),
        ],
        out_specs=pl.BlockSpec((DR, N), lambda i: (i, 0)),
        out_shape=jax.ShapeDtypeStruct((N, N), jnp.float32),
    )(mu, mu)


def kernel(x, edge_index, W_gcn, b_gcn, gamma1, beta1, W_mu, b_mu, W_lv, b_lv,
           W_proj, b_proj, gamma_p, beta_p, Wt1, bt1, Wt2, bt2):
    src = edge_index[0]
    dst = edge_index[1]
    pad = EPAD - E
    src_pad = jnp.concatenate([src, jnp.zeros((pad,), jnp.int32)])
    dst_pad = jnp.concatenate([dst, jnp.full((pad,), N, jnp.int32)])
    src2 = src_pad.reshape(EPAD // CHUNK, CHUNK)
    dst2 = dst_pad.reshape(EPAD // CHUNK, CHUNK)

    deg_sc, msg_sc = _sc_kernels()
    hist = deg_sc(dst2)
    hs, dinv = _gcn_pre(x, W_gcn, hist)
    acc = msg_sc(src2, dst2, hs)
    agg, ssum, ssq = _agg(acc, hs, dinv, b_gcn.reshape(1, HID))
    mu, logvar, zp, zsum, zsq = _enc(
        agg, ssum, ssq, gamma1.reshape(1, HID), beta1.reshape(1, HID),
        W_mu, b_mu.reshape(1, LZ), W_lv, b_lv.reshape(1, LZ),
        W_proj, b_proj.reshape(1, LT))
    x_recon = _teach(zp, zsum, zsq, gamma_p.reshape(1, LT), beta_p.reshape(1, LT),
                     Wt1, bt1.reshape(1, THID), Wt2, bt2.reshape(1, FIN))
    adj_recon = _adj(mu)
    return (adj_recon, x_recon, mu, logvar)


# speedup vs baseline: 1.0515x; 1.0515x over previous
import functools

import jax
import jax.numpy as jnp
from jax import lax
from jax.experimental import pallas as pl
from jax.experimental.pallas import tpu as pltpu
from jax.experimental.pallas import tpu_sc as plsc

N = 10000
E = 320000
FIN = 128
HID = 128
LZ = 32
LT = 64
THID = 256

NROW = 10240
EPAD = 327680
NW = 32
CHUNK = 128
CPT = EPAD // (NW * CHUNK)
RPT = NROW // 16
ZROWS = 64

def _deg_body(dst_hbm, out_hbm, ones, didx, ssem, hist):
    cid = lax.axis_index("c")
    sid = lax.axis_index("s")
    wid = sid * 2 + cid
    zero16 = jnp.zeros((16,), jnp.float32)
    ones16 = jnp.ones((16,), jnp.float32)
    base = sid * RPT

    pltpu.sync_copy(dst_hbm.at[pl.ds(wid * CPT, CPT)], didx)

    def _z(k, _):
        ones[k // (FIN // 16), pl.ds((k % (FIN // 16)) * 16, 16)] = zero16
        return 0

    lax.fori_loop(0, CHUNK * (FIN // 16), _z, 0)

    def _zc(k, _):
        pltpu.sync_copy(ones, hist.at[pl.ds(base + k * CHUNK, CHUNK)])
        return 0

    lax.fori_loop(0, RPT // CHUNK, _zc, 0)

    def _o(k, _):
        ones[k // (FIN // 16), pl.ds((k % (FIN // 16)) * 16, 16)] = ones16
        return 0

    lax.fori_loop(0, CHUNK * (FIN // 16), _o, 0)
    plsc.subcore_barrier()

    def _e(c, _):
        pltpu.async_copy(ones, hist.at[didx.at[c]], ssem, add=True)
        return 0

    lax.fori_loop(0, CPT, _e, 0)

    def _w(c, _):
        pltpu.make_async_copy(ones, hist.at[didx.at[c]], ssem).wait()
        return 0

    lax.fori_loop(0, CPT, _w, 0)
    plsc.subcore_barrier()
    pltpu.sync_copy(hist.at[pl.ds(base, RPT)], out_hbm.at[cid, pl.ds(base, RPT)])


G = 16
NGRP = CPT // G


def _msg_body(src_hbm, dst_hbm, hs_hbm, out_hbm, sidx, didx, rows,
              isem, g0, g1, ssem, acc):
    gsem = (g0, g1)
    cid = lax.axis_index("c")
    sid = lax.axis_index("s")
    wid = sid * 2 + cid
    zero16 = jnp.zeros((16,), jnp.float32)
    base = sid * RPT
    base0 = wid * CPT

    pltpu.sync_copy(src_hbm.at[pl.ds(base0, G)], sidx.at[0])
    pltpu.sync_copy(dst_hbm.at[pl.ds(base0, G)], didx.at[0])

    def _z(k, _):
        rows[0, k // (FIN // 16), pl.ds((k % (FIN // 16)) * 16, 16)] = zero16
        return 0

    lax.fori_loop(0, CHUNK * (FIN // 16), _z, 0)

    def _zc(k, _):
        pltpu.sync_copy(rows.at[0], acc.at[pl.ds(base + k * CHUNK, CHUNK)])
        return 0

    lax.fori_loop(0, RPT // CHUNK, _zc, 0)
    plsc.subcore_barrier()

    def _grp(g, _):
        slot = g % 2
        nxt = (g + 1) % 2

        @pl.when(g + 1 < NGRP)
        def _():
            pltpu.async_copy(
                src_hbm.at[pl.ds(base0 + (g + 1) * G, G)], sidx.at[nxt], isem)
            pltpu.async_copy(
                dst_hbm.at[pl.ds(base0 + (g + 1) * G, G)], didx.at[nxt], isem)

        for b in range(2):
            pltpu.async_copy(hs_hbm.at[sidx.at[slot, b]], rows.at[b], gsem[b])

        def _st(r, _):
            for b in range(2):
                k = r * 2 + b
                pltpu.make_async_copy(
                    hs_hbm.at[sidx.at[slot, k]], rows.at[b], gsem[b]).wait()
                pltpu.async_copy(
                    rows.at[b], acc.at[didx.at[slot, k]], ssem, add=True)
                pltpu.make_async_copy(
                    rows.at[b], acc.at[didx.at[slot, k]], ssem).wait()
                pltpu.async_copy(
                    hs_hbm.at[sidx.at[slot, k + 2]], rows.at[b], gsem[b])
            return 0

        lax.fori_loop(0, (G - 2) // 2, _st, 0)
        for b in range(2):
            k = G - 2 + b
            pltpu.make_async_copy(
                hs_hbm.at[sidx.at[slot, k]], rows.at[b], gsem[b]).wait()
            pltpu.async_copy(
                rows.at[b], acc.at[didx.at[slot, k]], ssem, add=True)
            pltpu.make_async_copy(
                rows.at[b], acc.at[didx.at[slot, k]], ssem).wait()

        @pl.when(g + 1 < NGRP)
        def _():
            pltpu.make_async_copy(
                src_hbm.at[pl.ds(base0, G)], sidx.at[nxt], isem).wait()
            pltpu.make_async_copy(
                dst_hbm.at[pl.ds(base0, G)], didx.at[nxt], isem).wait()
        return 0

    lax.fori_loop(0, NGRP, _grp, 0)
    plsc.subcore_barrier()
    pltpu.sync_copy(acc.at[pl.ds(base, RPT)], out_hbm.at[cid, pl.ds(base, RPT)])


@functools.cache
def _sc_kernels():
    mesh = plsc.VectorSubcoreMesh(
        core_axis_name="c", subcore_axis_name="s", num_cores=2, num_subcores=16)
    deg = pl.kernel(
        _deg_body,
        out_type=jax.ShapeDtypeStruct((2, NROW, FIN), jnp.float32),
        mesh=mesh,
        scratch_types=[
            pltpu.VMEM((CHUNK, FIN), jnp.float32),
            pltpu.VMEM((CPT, CHUNK), jnp.int32),
            pltpu.SemaphoreType.DMA,
            pltpu.VMEM_SHARED((NROW, FIN), jnp.float32),
        ],
    )
    msg = pl.kernel(
        _msg_body,
        out_type=jax.ShapeDtypeStruct((2, NROW, FIN), jnp.float32),
        mesh=mesh,
        scratch_types=[
            pltpu.VMEM((2, G, CHUNK), jnp.int32),
            pltpu.VMEM((2, G, CHUNK), jnp.int32),
            pltpu.VMEM((2, CHUNK, FIN), jnp.float32),
            pltpu.SemaphoreType.DMA,
            pltpu.SemaphoreType.DMA,
            pltpu.SemaphoreType.DMA,
            pltpu.SemaphoreType.DMA,
            pltpu.VMEM_SHARED((NROW, FIN), jnp.float32),
        ],
    )
    return deg, msg


BR = 1000


def _gcn_pre_body(x_ref, w_ref, hist_ref, hs_ref, dinv_ref):
    hb = hist_ref[...]
    deg = hb[0, :, 0:1] + hb[1, :, 0:1] + 1.0
    dinv = lax.rsqrt(jnp.maximum(deg, 1.0))
    h = jnp.dot(x_ref[...], w_ref[...], preferred_element_type=jnp.float32)
    hs_ref[...] = h * dinv
    dinv_ref[...] = dinv


def _gcn_pre(x, w, hist):
    return pl.pallas_call(
        _gcn_pre_body,
        grid=(N // BR,),
        in_specs=[
            pl.BlockSpec((BR, FIN), lambda i: (i, 0)),
            pl.BlockSpec((FIN, HID), lambda i: (0, 0)),
            pl.BlockSpec((2, BR, FIN), lambda i: (0, i, 0)),
        ],
        out_specs=[
            pl.BlockSpec((BR, HID), lambda i: (i, 0)),
            pl.BlockSpec((BR, 1), lambda i: (i, 0)),
        ],
        out_shape=[
            jax.ShapeDtypeStruct((N, HID), jnp.float32),
            jax.ShapeDtypeStruct((N, 1), jnp.float32),
        ],
    )(x, w, hist)


def _agg_body(acc_ref, hs_ref, dinv_ref, b_ref, agg_ref, s_ref, q_ref):
    a = acc_ref[...]
    t = a[0] + a[1] + hs_ref[...]
    agg = t * dinv_ref[...] + b_ref[...]
    agg_ref[...] = agg
    ps = jnp.sum(agg, axis=0, keepdims=True)
    pq = jnp.sum(agg * agg, axis=0, keepdims=True)
    i = pl.program_id(0)

    @pl.when(i == 0)
    def _():
        s_ref[...] = ps
        q_ref[...] = pq

    @pl.when(i != 0)
    def _():
        s_ref[...] += ps
        q_ref[...] += pq


def _agg(acc, hs, dinv, b_gcn):
    return pl.pallas_call(
        _agg_body,
        grid=(N // BR,),
        in_specs=[
            pl.BlockSpec((2, BR, HID), lambda i: (0, i, 0)),
            pl.BlockSpec((BR, HID), lambda i: (i, 0)),
            pl.BlockSpec((BR, 1), lambda i: (i, 0)),
            pl.BlockSpec((1, HID), lambda i: (0, 0)),
        ],
        out_specs=[
            pl.BlockSpec((BR, HID), lambda i: (i, 0)),
            pl.BlockSpec((1, HID), lambda i: (0, 0)),
            pl.BlockSpec((1, HID), lambda i: (0, 0)),
        ],
        out_shape=[
            jax.ShapeDtypeStruct((N, HID), jnp.float32),
            jax.ShapeDtypeStruct((1, HID), jnp.float32),
            jax.ShapeDtypeStruct((1, HID), jnp.float32),
        ],
    )(acc, hs, dinv, b_gcn)


def _enc_body(agg_ref, s_ref, q_ref, g1_ref, b1_ref, wmu_ref, bmu_ref,
              wlv_ref, blv_ref, wp_ref, bp_ref,
              mu_ref, lv_ref, zp_ref, zs_ref, zq_ref):
    inv_n = 1.0 / N
    mean = s_ref[...] * inv_n
    var = q_ref[...] * inv_n - mean * mean
    scale = g1_ref[...] * lax.rsqrt(var + 1e-5)
    hact = jnp.maximum((agg_ref[...] - mean) * scale + b1_ref[...], 0.0)
    mu = jnp.dot(hact, wmu_ref[...], preferred_element_type=jnp.float32) + bmu_ref[...]
    mu_ref[...] = mu
    lv_ref[...] = jnp.dot(hact, wlv_ref[...], preferred_element_type=jnp.float32) + blv_ref[...]
    zp = jnp.dot(mu, wp_ref[...], preferred_element_type=jnp.float32) + bp_ref[...]
    zp_ref[...] = zp
    ps = jnp.sum(zp, axis=0, keepdims=True)
    pq = jnp.sum(zp * zp, axis=0, keepdims=True)
    i = pl.program_id(0)

    @pl.when(i == 0)
    def _():
        zs_ref[...] = ps
        zq_ref[...] = pq

    @pl.when(i != 0)
    def _():
        zs_ref[...] += ps
        zq_ref[...] += pq


def _enc(agg, ssum, ssq, gamma1, beta1, w_mu, b_mu, w_lv, b_lv, w_p, b_p):
    return pl.pallas_call(
        _enc_body,
        grid=(N // BR,),
        in_specs=[
            pl.BlockSpec((BR, HID), lambda i: (i, 0)),
            pl.BlockSpec((1, HID), lambda i: (0, 0)),
            pl.BlockSpec((1, HID), lambda i: (0, 0)),
            pl.BlockSpec((1, HID), lambda i: (0, 0)),
            pl.BlockSpec((1, HID), lambda i: (0, 0)),
            pl.BlockSpec((HID, LZ), lambda i: (0, 0)),
            pl.BlockSpec((1, LZ), lambda i: (0, 0)),
            pl.BlockSpec((HID, LZ), lambda i: (0, 0)),
            pl.BlockSpec((1, LZ), lambda i: (0, 0)),
            pl.BlockSpec((LZ, LT), lambda i: (0, 0)),
            pl.BlockSpec((1, LT), lambda i: (0, 0)),
        ],
        out_specs=[
            pl.BlockSpec((BR, LZ), lambda i: (i, 0)),
            pl.BlockSpec((BR, LZ), lambda i: (i, 0)),
            pl.BlockSpec((BR, LT), lambda i: (i, 0)),
            pl.BlockSpec((1, LT), lambda i: (0, 0)),
            pl.BlockSpec((1, LT), lambda i: (0, 0)),
        ],
        out_shape=[
            jax.ShapeDtypeStruct((N, LZ), jnp.float32),
            jax.ShapeDtypeStruct((N, LZ), jnp.float32),
            jax.ShapeDtypeStruct((N, LT), jnp.float32),
            jax.ShapeDtypeStruct((1, LT), jnp.float32),
            jax.ShapeDtypeStruct((1, LT), jnp.float32),
        ],
    )(agg, ssum, ssq, gamma1, beta1, w_mu, b_mu, w_lv, b_lv, w_p, b_p)


def _teach_body(zp_ref, zs_ref, zq_ref, gp_ref, bp_ref, wt1_ref, bt1_ref,
                wt2_ref, bt2_ref, xr_ref):
    inv_n = 1.0 / N
    mean = zs_ref[...] * inv_n
    var = zq_ref[...] * inv_n - mean * mean
    scale = gp_ref[...] * lax.rsqrt(var + 1e-5)
    zpn = (zp_ref[...] - mean) * scale + bp_ref[...]
    t = jnp.maximum(
        jnp.dot(zpn, wt1_ref[...], preferred_element_type=jnp.float32) + bt1_ref[...], 0.0)
    xr_ref[...] = jnp.dot(t, wt2_ref[...], preferred_element_type=jnp.float32) + bt2_ref[...]


def _teach(zp, zsum, zsq, gamma_p, beta_p, wt1, bt1, wt2, bt2):
    return pl.pallas_call(
        _teach_body,
        grid=(N // BR,),
        in_specs=[
            pl.BlockSpec((BR, LT), lambda i: (i, 0)),
            pl.BlockSpec((1, LT), lambda i: (0, 0)),
            pl.BlockSpec((1, LT), lambda i: (0, 0)),
            pl.BlockSpec((1, LT), lambda i: (0, 0)),
            pl.BlockSpec((1, LT), lambda i: (0, 0)),
            pl.BlockSpec((LT, THID), lambda i: (0, 0)),
            pl.BlockSpec((1, THID), lambda i: (0, 0)),
            pl.BlockSpec((THID, FIN), lambda i: (0, 0)),
            pl.BlockSpec((1, FIN), lambda i: (0, 0)),
        ],
        out_specs=pl.BlockSpec((BR, FIN), lambda i: (i, 0)),
        out_shape=jax.ShapeDtypeStruct((N, FIN), jnp.float32),
    )(zp, zsum, zsq, gamma_p, beta_p, wt1, bt1, wt2, bt2)


DR = 400


def _adj_body(zi_ref, zj_ref, out_ref):
    d = lax.dot_general(zi_ref[...], zj_ref[...], (((1,), (1,)), ((), ())),
                        preferred_element_type=jnp.float32)
    out_ref[...] = jax.nn.sigmoid(d)


def _adj(mu):
    return pl.pallas_call(
        _adj_body,
        grid=(N // DR,),
        in_specs=[
            pl.BlockSpec((DR, LZ), lambda i: (i, 0)),
            pl.BlockSpec((N, LZ), lambda i: (0, 0)),
        ],
        out_specs=pl.BlockSpec((DR, N), lambda i: (i, 0)),
        out_shape=jax.ShapeDtypeStruct((N, N), jnp.float32),
    )(mu, mu)


def kernel(x, edge_index, W_gcn, b_gcn, gamma1, beta1, W_mu, b_mu, W_lv, b_lv,
           W_proj, b_proj, gamma_p, beta_p, Wt1, bt1, Wt2, bt2):
    src = edge_index[0]
    dst = edge_index[1]
    pad = EPAD - E
    src_pad = jnp.concatenate([src, jnp.zeros((pad,), jnp.int32)])
    dst_pad = jnp.concatenate([dst, jnp.full((pad,), N, jnp.int32)])
    src2 = src_pad.reshape(EPAD // CHUNK, CHUNK)
    dst2 = dst_pad.reshape(EPAD // CHUNK, CHUNK)

    deg_sc, msg_sc = _sc_kernels()
    hist = deg_sc(dst2)
    hs, dinv = _gcn_pre(x, W_gcn, hist)
    acc = msg_sc(src2, dst2, hs)
    agg, ssum, ssq = _agg(acc, hs, dinv, b_gcn.reshape(1, HID))
    mu, logvar, zp, zsum, zsq = _enc(
        agg, ssum, ssq, gamma1.reshape(1, HID), beta1.reshape(1, HID),
        W_mu, b_mu.reshape(1, LZ), W_lv, b_lv.reshape(1, LZ),
        W_proj, b_proj.reshape(1, LT))
    x_recon = _teach(zp, zsum, zsq, gamma_p.reshape(1, LT), beta_p.reshape(1, LT),
                     Wt1, bt1.reshape(1, THID), Wt2, bt2.reshape(1, FIN))
    adj_recon = _adj(mu)
    return (adj_recon, x_recon, mu, logvar)
